# baseline (speedup 1.0000x reference)
import jax
import jax.numpy as jnp
from jax.experimental import pallas as pl

N_CLS = 1000
ROWS = 4096 * 20
BLK = 256


def _onehot_block(x_ref, o_ref):
    ids = x_ref[...]
    cols = jax.lax.broadcasted_iota(jnp.int32, (BLK, N_CLS), 1)
    o_ref[...] = (ids == cols).astype(jnp.float32)


def kernel(x):
    xf = x.reshape(ROWS, 1).astype(jnp.int32)
    out = pl.pallas_call(
        _onehot_block,
        grid=(ROWS // BLK,),
        in_specs=[pl.BlockSpec((BLK, 1), lambda i: (i, 0))],
        out_specs=pl.BlockSpec((BLK, N_CLS), lambda i: (i, 0)),
        out_shape=jax.ShapeDtypeStruct((ROWS, N_CLS), jnp.float32),
    )(xf)
    return out.reshape(4096, 20, N_CLS)


# speedup vs baseline: 1.2200x; 1.2200x over previous
import functools

import jax
import jax.numpy as jnp
from jax import lax
from jax.experimental import pallas as pl
from jax.experimental.pallas import tpu as pltpu
from jax.experimental.pallas import tpu_sc as plsc

N_CLS = 1000
ROWS = 4096 * 20
NC, NS, L = 2, 16, 16
NW = NC * NS
R_PER_W = ROWS // NW
CHUNK = 64
N_CHUNKS = R_PER_W // CHUNK


def _sc_onehot(x_hbm, out_hbm, idx_v, buf_v):
    wid = lax.axis_index("s") * NC + lax.axis_index("c")
    row0 = wid * R_PER_W

    pltpu.sync_copy(x_hbm.at[pl.ds(row0 * 1, R_PER_W)], idx_v)

    def _zero(i, _):
        buf_v[pl.ds(i * L, L)] = jnp.zeros((L,), jnp.float32)
        return 0
    lax.fori_loop(0, (CHUNK * N_CLS) // L, _zero, 0)

    lane = lax.iota(jnp.int32, L) * N_CLS
    ones = jnp.full((L,), 1.0, jnp.float32)
    zeros = jnp.zeros((L,), jnp.float32)

    def _chunk(g, _):
        flats = []
        for j in range(CHUNK // L):
            cols = idx_v[pl.ds(g * CHUNK + j * L, L)]
            flats.append(lane + (j * L * N_CLS) + cols)
        for f in flats:
            plsc.store_scatter(buf_v, [f], ones)
        pltpu.sync_copy(
            buf_v, out_hbm.at[pl.ds((row0 + g * CHUNK) * N_CLS, CHUNK * N_CLS)]
        )
        for f in flats:
            plsc.store_scatter(buf_v, [f], zeros)
        return 0

    lax.fori_loop(0, N_CHUNKS, _chunk, 0)


def kernel(x):
    xf = x.reshape(ROWS).astype(jnp.int32)
    mesh = plsc.VectorSubcoreMesh(core_axis_name="c", subcore_axis_name="s")
    out = pl.kernel(
        _sc_onehot,
        out_type=jax.ShapeDtypeStruct((ROWS * N_CLS,), jnp.float32),
        mesh=mesh,
        scratch_types=[
            pltpu.VMEM((R_PER_W,), jnp.int32),
            pltpu.VMEM((CHUNK * N_CLS,), jnp.float32),
        ],
        compiler_params=pltpu.CompilerParams(needs_layout_passes=False),
    )(xf)
    return out.reshape(4096, 20, N_CLS)
